# baseline (speedup 1.0000x reference)
import jax
import jax.numpy as jnp
from jax import lax
from jax.experimental import pallas as pl
from jax.experimental.pallas import tpu as pltpu
from jax.experimental.pallas import tpu_sc as plsc

N = 10000
IN_DIM = 128
H = 16
E = 320000

NCORES = 2
NTILES = 16
NW = NCORES * NTILES
CHUNK = 128
NCHUNKS = E // CHUNK
CMIN1 = NCHUNKS // NTILES
CREM1 = NCHUNKS % NTILES
CMIN2 = NCHUNKS // NW
CREM2 = NCHUNKS % NW
NSLICE = 640
NB = 4

_SC_PARAMS = pltpu.CompilerParams(
    needs_layout_passes=False, use_tc_tiling_on_sc=False)


def _mm_body(x_ref, w_ref, o_ref):
    o_ref[...] = jnp.dot(x_ref[...], w_ref[...],
                         preferred_element_type=jnp.float32)


def _matmul(x, W1):
    return pl.pallas_call(
        _mm_body,
        out_shape=jax.ShapeDtypeStruct((N, H), jnp.float32),
        grid=(5,),
        in_specs=[
            pl.BlockSpec((2000, IN_DIM), lambda i: (i, 0)),
            pl.BlockSpec((IN_DIM, H), lambda i: (0, 0)),
        ],
        out_specs=pl.BlockSpec((2000, H), lambda i: (i, 0)),
    )(x, W1)


def _rsqrt16(d):
    ih = plsc.bitcast(d, jnp.int32)
    y = plsc.bitcast(jnp.int32(0x5F3759DF) - (ih >> 1), jnp.float32)
    hd = 0.5 * d
    y = y * (1.5 - hd * y * y)
    y = y * (1.5 - hd * y * y)
    y = y * (1.5 - hd * y * y)
    return y


def _stage_edges(ei_hbm, ei_v, cbase, cmin, has_tail):
    pltpu.sync_copy(ei_hbm.at[pl.ds(cbase, cmin)], ei_v.at[pl.ds(0, cmin)])

    @pl.when(has_tail)
    def _tail():
        pltpu.sync_copy(ei_hbm.at[pl.ds(cbase + cmin, 1)],
                        ei_v.at[pl.ds(cmin, 1)])


def _edge_pass(ei_v, cnt, table_sh, acc_sh, bufs, gsems, ssems):
    pltpu.async_copy(table_sh.at[ei_v.at[0, 0]], bufs.at[0], gsems.at[0])
    pltpu.async_copy(table_sh.at[ei_v.at[1, 0]], bufs.at[1], gsems.at[1])

    def step(j, c):
        b = lax.rem(j, NB)
        bn = lax.rem(j + 2, NB)

        @pl.when(j + 2 < cnt)
        def _prefetch():
            @pl.when(j >= 2)
            def _drain():
                pltpu.make_async_copy(
                    bufs.at[bn], acc_sh.at[ei_v.at[j - 2, 1]],
                    ssems.at[bn]).wait()
            pltpu.async_copy(
                table_sh.at[ei_v.at[j + 2, 0]], bufs.at[bn], gsems.at[bn])

        pltpu.make_async_copy(
            table_sh.at[ei_v.at[j, 0]], bufs.at[b], gsems.at[b]).wait()
        pltpu.async_copy(bufs.at[b], acc_sh.at[ei_v.at[j, 1]], ssems.at[b],
                         add=True)
        return c

    lax.fori_loop(0, cnt, step, 0)

    def drain(j, c):
        b = lax.rem(j, NB)
        pltpu.make_async_copy(
            bufs.at[b], acc_sh.at[ei_v.at[j, 1]], ssems.at[b]).wait()
        return c
    lax.fori_loop(cnt - NB, cnt, drain, 0)


def _deg_body(ei_hbm, degp_hbm,
              deg_sh, ei_v, ones_v, sc_v, dsems):
    cid = lax.axis_index("c")
    sid = lax.axis_index("s")
    wid = cid * NTILES + sid
    nbase = lax.min(sid * NSLICE, N - NSLICE)
    cbase = CMIN2 * wid + lax.min(wid, CREM2)
    cnt = CMIN2 + jnp.where(wid < CREM2, 1, 0)
    _stage_edges(ei_hbm, ei_v, cbase, CMIN2, wid < CREM2)
    for i in range(CHUNK // 16):
        ones_v[pl.ds(i * 16, 16)] = jnp.ones((16,), jnp.float32)

    def _zero(i, c):
        sc_v[pl.ds(i * 16, 16)] = jnp.zeros((16,), jnp.float32)
        return c
    lax.fori_loop(0, NSLICE // 16, _zero, 0)
    pltpu.sync_copy(sc_v, deg_sh.at[pl.ds(nbase, NSLICE)])
    plsc.subcore_barrier()

    def _deg(j, c):
        b = lax.rem(j, NB)

        @pl.when(j >= NB)
        def _drain():
            pltpu.make_async_copy(
                ones_v, deg_sh.at[ei_v.at[j - NB, 1]], dsems.at[b]).wait()
        pltpu.async_copy(ones_v, deg_sh.at[ei_v.at[j, 1]], dsems.at[b],
                         add=True)
        return c
    lax.fori_loop(0, cnt, _deg, 0)

    def _ddrain(j, c):
        pltpu.make_async_copy(
            ones_v, deg_sh.at[ei_v.at[j, 1]], dsems.at[lax.rem(j, NB)]).wait()
        return c
    lax.fori_loop(cnt - NB, cnt, _ddrain, 0)
    plsc.subcore_barrier()

    pltpu.sync_copy(deg_sh.at[pl.ds(nbase, NSLICE)], sc_v)
    pltpu.sync_copy(sc_v, degp_hbm.at[cid, pl.ds(nbase, NSLICE)])


_deg_sc = pl.kernel(
    _deg_body,
    out_type=jax.ShapeDtypeStruct((NCORES, N), jnp.float32),
    mesh=plsc.VectorSubcoreMesh(
        core_axis_name="c", subcore_axis_name="s", num_cores=2),
    scratch_types=[
        pltpu.VMEM_SHARED((N,), jnp.float32),
        pltpu.VMEM((CMIN2 + 1, 2, CHUNK), jnp.int32),
        pltpu.VMEM((CHUNK,), jnp.float32),
        pltpu.VMEM((NSLICE,), jnp.float32),
        pltpu.SemaphoreType.DMA((NB,)),
    ],
    compiler_params=_SC_PARAMS,
)


def _ka_body(h0_hbm, ei_hbm, degp_hbm, dinv_hbm, acc1p_hbm,
             g_sh, acc1_sh,
             ei_v, rows4_v, nodes_v, half_v, dinv_v, sc_v, gsems, ssems):
    cid = lax.axis_index("c")
    sid = lax.axis_index("s")
    wid = cid * NTILES + sid
    nbase = lax.min(sid * NSLICE, N - NSLICE)
    cbase = CMIN2 * wid + lax.min(wid, CREM2)
    cnt = CMIN2 + jnp.where(wid < CREM2, 1, 0)

    _stage_edges(ei_hbm, ei_v, cbase, CMIN2, wid < CREM2)
    pltpu.sync_copy(degp_hbm.at[0, pl.ds(nbase, NSLICE)], sc_v)
    pltpu.sync_copy(degp_hbm.at[1, pl.ds(nbase, NSLICE)], dinv_v)
    def _dinv(i, c):
        d = sc_v[pl.ds(i * 16, 16)] + dinv_v[pl.ds(i * 16, 16)] + 1.0
        dinv_v[pl.ds(i * 16, 16)] = _rsqrt16(d)
        return c
    lax.fori_loop(0, NSLICE // 16, _dinv, 0)

    @pl.when(cid == 0)
    def _wdinv():
        pltpu.sync_copy(dinv_v, dinv_hbm.at[pl.ds(nbase, NSLICE)])

    pltpu.sync_copy(h0_hbm.at[pl.ds(nbase, NSLICE)], nodes_v)
    iota = lax.iota(jnp.int32, 16)
    def _scale(gi, c):
        dv = dinv_v[pl.ds(gi * 16, 16)]
        rows = gi * 16 + iota
        for k in range(H):
            cols = jnp.full((16,), k, jnp.int32)
            col = plsc.load_gather(nodes_v, [rows, cols]) * dv
            plsc.store_scatter(nodes_v, [rows, cols], col)
            plsc.store_scatter(half_v, [rows, cols], col * 0.5)
        return c
    lax.fori_loop(0, NSLICE // 16, _scale, 0)
    pltpu.sync_copy(nodes_v, g_sh.at[pl.ds(nbase, NSLICE)])
    pltpu.sync_copy(half_v, acc1_sh.at[pl.ds(nbase, NSLICE)])
    plsc.subcore_barrier()

    _edge_pass(ei_v, cnt, g_sh, acc1_sh, rows4_v, gsems, ssems)
    plsc.subcore_barrier()

    pltpu.sync_copy(acc1_sh.at[pl.ds(nbase, NSLICE)], nodes_v)
    pltpu.sync_copy(nodes_v, acc1p_hbm.at[cid, pl.ds(nbase, NSLICE)])


_ka_sc = pl.kernel(
    _ka_body,
    out_type=(jax.ShapeDtypeStruct((N,), jnp.float32),
              jax.ShapeDtypeStruct((NCORES, N, H), jnp.float32)),
    mesh=plsc.VectorSubcoreMesh(
        core_axis_name="c", subcore_axis_name="s", num_cores=2),
    scratch_types=[
        pltpu.VMEM_SHARED((N, H), jnp.float32),
        pltpu.VMEM_SHARED((N, H), jnp.float32),
        pltpu.VMEM((CMIN2 + 1, 2, CHUNK), jnp.int32),
        pltpu.VMEM((NB, CHUNK, H), jnp.float32),
        pltpu.VMEM((NSLICE, H), jnp.float32),
        pltpu.VMEM((NSLICE, H), jnp.float32),
        pltpu.VMEM((NSLICE,), jnp.float32),
        pltpu.VMEM((NSLICE,), jnp.float32),
        pltpu.SemaphoreType.DMA((NB,)),
        pltpu.SemaphoreType.DMA((NB,)),
    ],
    compiler_params=_SC_PARAMS,
)


def _kb_body(ei_hbm, acc1p_hbm, dinv_hbm, b1_hbm, w2_hbm, b2_hbm,
             out_hbm,
             gs_sh, acc2_sh,
             ei_v, svals4_v, p0_v, p1_v, dinv_v,
             sc_v, out_v, b1_v, w2_v, b2_v, gsems, ssems):
    sid = lax.axis_index("s")
    nbase = lax.min(sid * NSLICE, N - NSLICE)
    cbase = CMIN1 * sid + lax.min(sid, CREM1)
    cnt = CMIN1 + jnp.where(sid < CREM1, 1, 0)

    _stage_edges(ei_hbm, ei_v, cbase, CMIN1, sid < CREM1)
    pltpu.sync_copy(dinv_hbm.at[pl.ds(nbase, NSLICE)], dinv_v)
    pltpu.sync_copy(b1_hbm, b1_v)
    pltpu.sync_copy(w2_hbm, w2_v)
    pltpu.sync_copy(b2_hbm, b2_v)
    pltpu.sync_copy(acc1p_hbm.at[0, pl.ds(nbase, NSLICE)], p0_v)
    pltpu.sync_copy(acc1p_hbm.at[1, pl.ds(nbase, NSLICE)], p1_v)

    iota = lax.iota(jnp.int32, 16)
    b1vec = b1_v[...]
    w2vec = w2_v[...]
    def _proj(gi, c):
        dv = dinv_v[pl.ds(gi * 16, 16)]
        acc = jnp.zeros((16,), jnp.float32)
        rows = gi * 16 + iota
        for k in range(H):
            cols = jnp.full((16,), k, jnp.int32)
            col = (plsc.load_gather(p0_v, [rows, cols]) +
                   plsc.load_gather(p1_v, [rows, cols]))
            hk = jnp.maximum(col * dv + b1vec[k], 0.0)
            acc = acc + hk * w2vec[k]
        sc_v[pl.ds(gi * 16, 16)] = acc * dv
        return c
    lax.fori_loop(0, NSLICE // 16, _proj, 0)
    pltpu.sync_copy(sc_v, gs_sh.at[pl.ds(nbase, NSLICE)])
    pltpu.sync_copy(sc_v, acc2_sh.at[pl.ds(nbase, NSLICE)])
    plsc.subcore_barrier()

    _edge_pass(ei_v, cnt, gs_sh, acc2_sh, svals4_v, gsems, ssems)
    plsc.subcore_barrier()

    pltpu.sync_copy(acc2_sh.at[pl.ds(nbase, NSLICE)], sc_v)
    b2vec = b2_v[...]
    def _out(i, c):
        v = sc_v[pl.ds(i * 16, 16)] * dinv_v[pl.ds(i * 16, 16)] + b2vec
        out_v[pl.ds(i * 16, 16)] = v
        return c
    lax.fori_loop(0, NSLICE // 16, _out, 0)
    pltpu.sync_copy(out_v, out_hbm.at[pl.ds(nbase, NSLICE)])


_kb_sc = pl.kernel(
    _kb_body,
    out_type=jax.ShapeDtypeStruct((N,), jnp.float32),
    mesh=plsc.VectorSubcoreMesh(
        core_axis_name="c", subcore_axis_name="s", num_cores=1),
    scratch_types=[
        pltpu.VMEM_SHARED((N,), jnp.float32),
        pltpu.VMEM_SHARED((N,), jnp.float32),
        pltpu.VMEM((CMIN1 + 1, 2, CHUNK), jnp.int32),
        pltpu.VMEM((NB, CHUNK), jnp.float32),
        pltpu.VMEM((NSLICE, H), jnp.float32),
        pltpu.VMEM((NSLICE, H), jnp.float32),
        pltpu.VMEM((NSLICE,), jnp.float32),
        pltpu.VMEM((NSLICE,), jnp.float32),
        pltpu.VMEM((NSLICE,), jnp.float32),
        pltpu.VMEM((16,), jnp.float32),
        pltpu.VMEM((16,), jnp.float32),
        pltpu.VMEM((16,), jnp.float32),
        pltpu.SemaphoreType.DMA((NB,)),
        pltpu.SemaphoreType.DMA((NB,)),
    ],
    compiler_params=_SC_PARAMS,
)


@jax.jit
def kernel(x, edge_index, W1, b1, W2, b2):
    ei3 = edge_index.reshape(2, NCHUNKS, CHUNK).transpose(1, 0, 2)
    degp = _deg_sc(ei3)
    h0 = _matmul(x, W1)
    dinv, acc1p = _ka_sc(h0, ei3, degp)
    b1h = b1.astype(jnp.float32)
    w2h = W2.reshape(H).astype(jnp.float32)
    b2h = jnp.full((16,), b2[0], jnp.float32)
    return _kb_sc(ei3, acc1p, dinv, b1h, w2h, b2h)


# speedup vs baseline: 1.0496x; 1.0496x over previous
import jax
import jax.numpy as jnp
from jax import lax
from jax.experimental import pallas as pl
from jax.experimental.pallas import tpu as pltpu
from jax.experimental.pallas import tpu_sc as plsc

N = 10000
IN_DIM = 128
H = 16
E = 320000

NTILES = 16
CHUNK = 128
NCHUNKS = E // CHUNK
CMIN = NCHUNKS // NTILES
CREM = NCHUNKS % NTILES
CMIN2 = NCHUNKS // 32
CREM2 = NCHUNKS % 32
NSLICE = 640
NB = 4

_SC_PARAMS = pltpu.CompilerParams(
    needs_layout_passes=False, use_tc_tiling_on_sc=False)


def _mm_body(x_ref, w_ref, o_ref):
    o_ref[...] = jnp.dot(x_ref[...], w_ref[...],
                         preferred_element_type=jnp.float32)


def _matmul(x, W1):
    return pl.pallas_call(
        _mm_body,
        out_shape=jax.ShapeDtypeStruct((N, H), jnp.float32),
        grid=(5,),
        in_specs=[
            pl.BlockSpec((2000, IN_DIM), lambda i: (i, 0)),
            pl.BlockSpec((IN_DIM, H), lambda i: (0, 0)),
        ],
        out_specs=pl.BlockSpec((2000, H), lambda i: (i, 0)),
    )(x, W1)


def _rsqrt16(d):
    ih = plsc.bitcast(d, jnp.int32)
    y = plsc.bitcast(jnp.int32(0x5F3759DF) - (ih >> 1), jnp.float32)
    hd = 0.5 * d
    y = y * (1.5 - hd * y * y)
    y = y * (1.5 - hd * y * y)
    y = y * (1.5 - hd * y * y)
    return y


def _chunk_range(sid):
    cnt = CMIN + jnp.where(sid < CREM, 1, 0)
    cbase = CMIN * sid + lax.min(sid, CREM)
    return cbase, cnt


def _stage_edges(ei_hbm, ei_v, sid):
    cbase, cnt = _chunk_range(sid)
    pltpu.sync_copy(ei_hbm.at[pl.ds(cbase, CMIN)], ei_v.at[pl.ds(0, CMIN)])

    @pl.when(sid < CREM)
    def _tail():
        pltpu.sync_copy(ei_hbm.at[pl.ds(cbase + CMIN, 1)],
                        ei_v.at[pl.ds(CMIN, 1)])
    return cnt


def _edge_pass(ei_v, cnt, table_sh, acc_sh, bufs, gsems, ssems):
    pltpu.async_copy(table_sh.at[ei_v.at[0, 0]], bufs.at[0], gsems.at[0])
    pltpu.async_copy(table_sh.at[ei_v.at[1, 0]], bufs.at[1], gsems.at[1])

    def step(j, c):
        b = lax.rem(j, NB)
        bn = lax.rem(j + 2, NB)

        @pl.when(j + 2 < cnt)
        def _prefetch():
            @pl.when(j >= 2)
            def _drain():
                pltpu.make_async_copy(
                    bufs.at[bn], acc_sh.at[ei_v.at[j - 2, 1]],
                    ssems.at[bn]).wait()
            pltpu.async_copy(
                table_sh.at[ei_v.at[j + 2, 0]], bufs.at[bn], gsems.at[bn])

        pltpu.make_async_copy(
            table_sh.at[ei_v.at[j, 0]], bufs.at[b], gsems.at[b]).wait()
        pltpu.async_copy(bufs.at[b], acc_sh.at[ei_v.at[j, 1]], ssems.at[b],
                         add=True)
        return c

    lax.fori_loop(0, cnt, step, 0)

    def drain(j, c):
        b = lax.rem(j, NB)
        pltpu.make_async_copy(
            bufs.at[b], acc_sh.at[ei_v.at[j, 1]], ssems.at[b]).wait()
        return c
    lax.fori_loop(cnt - NB, cnt, drain, 0)


def _deg_body(ei_hbm, degp_hbm,
              deg_sh, ei_v, ones_v, sc_v, dsems):
    cid = lax.axis_index("c")
    sid = lax.axis_index("s")
    wid = cid * NTILES + sid
    nbase = lax.min(sid * NSLICE, N - NSLICE)
    cbase = CMIN2 * wid + lax.min(wid, CREM2)
    cnt = CMIN2 + jnp.where(wid < CREM2, 1, 0)
    pltpu.sync_copy(ei_hbm.at[pl.ds(cbase, CMIN2)], ei_v.at[pl.ds(0, CMIN2)])

    @pl.when(wid < CREM2)
    def _tail_stage():
        pltpu.sync_copy(ei_hbm.at[pl.ds(cbase + CMIN2, 1)],
                        ei_v.at[pl.ds(CMIN2, 1)])
    for i in range(CHUNK // 16):
        ones_v[pl.ds(i * 16, 16)] = jnp.ones((16,), jnp.float32)

    def _zero(i, c):
        sc_v[pl.ds(i * 16, 16)] = jnp.zeros((16,), jnp.float32)
        return c
    lax.fori_loop(0, NSLICE // 16, _zero, 0)
    pltpu.sync_copy(sc_v, deg_sh.at[pl.ds(nbase, NSLICE)])
    plsc.subcore_barrier()

    def _deg(j, c):
        b = lax.rem(j, NB)

        @pl.when(j >= NB)
        def _drain():
            pltpu.make_async_copy(
                ones_v, deg_sh.at[ei_v.at[j - NB, 1]], dsems.at[b]).wait()
        pltpu.async_copy(ones_v, deg_sh.at[ei_v.at[j, 1]], dsems.at[b],
                         add=True)
        return c
    lax.fori_loop(0, cnt, _deg, 0)

    def _ddrain(j, c):
        pltpu.make_async_copy(
            ones_v, deg_sh.at[ei_v.at[j, 1]], dsems.at[lax.rem(j, NB)]).wait()
        return c
    lax.fori_loop(cnt - NB, cnt, _ddrain, 0)
    plsc.subcore_barrier()

    pltpu.sync_copy(deg_sh.at[pl.ds(nbase, NSLICE)], sc_v)
    pltpu.sync_copy(sc_v, degp_hbm.at[cid, pl.ds(nbase, NSLICE)])


_deg_sc = pl.kernel(
    _deg_body,
    out_type=jax.ShapeDtypeStruct((2, N), jnp.float32),
    mesh=plsc.VectorSubcoreMesh(
        core_axis_name="c", subcore_axis_name="s", num_cores=2),
    scratch_types=[
        pltpu.VMEM_SHARED((N,), jnp.float32),
        pltpu.VMEM((CMIN2 + 1, 2, CHUNK), jnp.int32),
        pltpu.VMEM((CHUNK,), jnp.float32),
        pltpu.VMEM((NSLICE,), jnp.float32),
        pltpu.SemaphoreType.DMA((NB,)),
    ],
    compiler_params=_SC_PARAMS,
)


def _gcn_body(h0_hbm, ei_hbm, degp_hbm, b1_hbm, w2_hbm, b2_hbm,
              out_hbm,
              g_sh, acc1_sh, gs_sh, acc2_sh,
              ei_v, rows4_v, svals4_v, nodes_v, dinv_v,
              sc_v, out_v, b1_v, w2_v, b2_v, gsems, ssems, esem):
    sid = lax.axis_index("s")
    nbase = lax.min(sid * NSLICE, N - NSLICE)
    cbase, cnt = _chunk_range(sid)

    pltpu.async_copy(ei_hbm.at[pl.ds(cbase, CMIN)],
                     ei_v.at[pl.ds(0, CMIN)], esem)

    @pl.when(sid < CREM)
    def _tail_fire():
        pltpu.async_copy(ei_hbm.at[pl.ds(cbase + CMIN, 1)],
                         ei_v.at[pl.ds(CMIN, 1)], esem)
    pltpu.sync_copy(b1_hbm, b1_v)
    pltpu.sync_copy(w2_hbm, w2_v)
    pltpu.sync_copy(b2_hbm, b2_v)

    pltpu.sync_copy(degp_hbm.at[0, pl.ds(nbase, NSLICE)], sc_v)
    pltpu.sync_copy(degp_hbm.at[1, pl.ds(nbase, NSLICE)], dinv_v)
    def _dinvm(i, c):
        d = sc_v[pl.ds(i * 16, 16)] + dinv_v[pl.ds(i * 16, 16)] + 1.0
        dinv_v[pl.ds(i * 16, 16)] = _rsqrt16(d)
        return c
    lax.fori_loop(0, NSLICE // 16, _dinvm, 0)

    pltpu.sync_copy(h0_hbm.at[pl.ds(nbase, NSLICE)], nodes_v)
    iota = lax.iota(jnp.int32, 16)
    def _scale(gi, c):
        dv = dinv_v[pl.ds(gi * 16, 16)]
        rows = gi * 16 + iota
        for k in range(H):
            cols = jnp.full((16,), k, jnp.int32)
            col = plsc.load_gather(nodes_v, [rows, cols])
            plsc.store_scatter(nodes_v, [rows, cols], col * dv)
        return c
    lax.fori_loop(0, NSLICE // 16, _scale, 0)
    pltpu.sync_copy(nodes_v, g_sh.at[pl.ds(nbase, NSLICE)])
    pltpu.sync_copy(nodes_v, acc1_sh.at[pl.ds(nbase, NSLICE)])
    plsc.subcore_barrier()

    pltpu.make_async_copy(ei_hbm.at[pl.ds(cbase, CMIN)],
                          ei_v.at[pl.ds(0, CMIN)], esem).wait()

    @pl.when(sid < CREM)
    def _tail_wait():
        pltpu.make_async_copy(ei_hbm.at[pl.ds(cbase + CMIN, 1)],
                              ei_v.at[pl.ds(CMIN, 1)], esem).wait()
    _edge_pass(ei_v, cnt, g_sh, acc1_sh, rows4_v, gsems, ssems)
    plsc.subcore_barrier()

    pltpu.sync_copy(acc1_sh.at[pl.ds(nbase, NSLICE)], nodes_v)
    b1vec = b1_v[...]
    w2vec = w2_v[...]
    def _proj(gi, c):
        dv = dinv_v[pl.ds(gi * 16, 16)]
        acc = jnp.zeros((16,), jnp.float32)
        rows = gi * 16 + iota
        for k in range(H):
            col = plsc.load_gather(
                nodes_v, [rows, jnp.full((16,), k, jnp.int32)])
            hk = jnp.maximum(col * dv + b1vec[k], 0.0)
            acc = acc + hk * w2vec[k]
        sc_v[pl.ds(gi * 16, 16)] = acc * dv
        return c
    lax.fori_loop(0, NSLICE // 16, _proj, 0)
    pltpu.sync_copy(sc_v, gs_sh.at[pl.ds(nbase, NSLICE)])
    pltpu.sync_copy(sc_v, acc2_sh.at[pl.ds(nbase, NSLICE)])
    plsc.subcore_barrier()

    _edge_pass(ei_v, cnt, gs_sh, acc2_sh, svals4_v, gsems, ssems)
    plsc.subcore_barrier()

    pltpu.sync_copy(acc2_sh.at[pl.ds(nbase, NSLICE)], sc_v)
    b2vec = b2_v[...]
    def _out(i, c):
        v = sc_v[pl.ds(i * 16, 16)] * dinv_v[pl.ds(i * 16, 16)] + b2vec
        out_v[pl.ds(i * 16, 16)] = v
        return c
    lax.fori_loop(0, NSLICE // 16, _out, 0)
    pltpu.sync_copy(out_v, out_hbm.at[pl.ds(nbase, NSLICE)])


_gcn_sc = pl.kernel(
    _gcn_body,
    out_type=jax.ShapeDtypeStruct((N,), jnp.float32),
    mesh=plsc.VectorSubcoreMesh(
        core_axis_name="c", subcore_axis_name="s", num_cores=1),
    scratch_types=[
        pltpu.VMEM_SHARED((N, H), jnp.float32),
        pltpu.VMEM_SHARED((N, H), jnp.float32),
        pltpu.VMEM_SHARED((N,), jnp.float32),
        pltpu.VMEM_SHARED((N,), jnp.float32),
        pltpu.VMEM((CMIN + 1, 2, CHUNK), jnp.int32),
        pltpu.VMEM((NB, CHUNK, H), jnp.float32),
        pltpu.VMEM((NB, CHUNK), jnp.float32),
        pltpu.VMEM((NSLICE, H), jnp.float32),
        pltpu.VMEM((NSLICE,), jnp.float32),
        pltpu.VMEM((NSLICE,), jnp.float32),
        pltpu.VMEM((NSLICE,), jnp.float32),
        pltpu.VMEM((16,), jnp.float32),
        pltpu.VMEM((16,), jnp.float32),
        pltpu.VMEM((16,), jnp.float32),
        pltpu.SemaphoreType.DMA((NB,)),
        pltpu.SemaphoreType.DMA((NB,)),
        pltpu.SemaphoreType.DMA,
    ],
    compiler_params=_SC_PARAMS,
)


@jax.jit
def kernel(x, edge_index, W1, b1, W2, b2):
    ei3 = edge_index.reshape(2, NCHUNKS, CHUNK).transpose(1, 0, 2)
    degp = _deg_sc(ei3)
    h0 = _matmul(x, W1)
    b1h = b1.astype(jnp.float32)
    w2h = W2.reshape(H).astype(jnp.float32)
    b2h = jnp.full((16,), b2[0], jnp.float32)
    return _gcn_sc(h0, ei3, degp, b1h, w2h, b2h)
